# baseline (speedup 1.0000x reference)
import jax
import jax.numpy as jnp
from jax.experimental import pallas as pl
from jax.experimental.pallas import tpu as pltpu

T, B, E, H, D = 200, 1024, 64, 2, 256
BB = 128
DC = 128


def _dnd_read_kernel(kt_ref, v_ref, rpe_ref, qt_ref, w_ref, b_ref,
                     out_ref, w0_ref, w1_ref):
    s = pl.program_id(1)

    @pl.when(s == 0)
    def _():
        kt = kt_ref[...]
        qt = qt_ref[...]
        r = rpe_ref[...]
        l0 = jnp.sum(kt * qt[:E][None], axis=1) * r
        l1 = jnp.sum(kt * qt[E:][None], axis=1) * r
        e0 = jnp.exp(l0 - jnp.max(l0, axis=0, keepdims=True))
        w0_ref[...] = e0 / jnp.sum(e0, axis=0, keepdims=True)
        e1 = jnp.exp(l1 - jnp.max(l1, axis=0, keepdims=True))
        w1_ref[...] = e1 / jnp.sum(e1, axis=0, keepdims=True)

    def _acc(t, carry):
        a0, a1 = carry
        vt = v_ref[t]
        w0c = jnp.transpose(w0_ref[pl.ds(t, 1), :])
        w1c = jnp.transpose(w1_ref[pl.ds(t, 1), :])
        return a0 + w0c * vt, a1 + w1c * vt

    zero = jnp.zeros((BB, DC), dtype=jnp.float32)
    res0, res1 = jax.lax.fori_loop(0, T, _acc, (zero, zero))

    wc0 = w_ref[:, pl.ds(s * DC, DC)]
    wc1 = w_ref[:, pl.ds(D + s * DC, DC)]
    part = (
        jax.lax.dot_general(res0, wc0, (((1,), (1,)), ((), ())),
                            preferred_element_type=jnp.float32)
        + jax.lax.dot_general(res1, wc1, (((1,), (1,)), ((), ())),
                              preferred_element_type=jnp.float32)
    )

    @pl.when(s == 0)
    def _():
        out_ref[...] = part + b_ref[...]

    @pl.when(s != 0)
    def _():
        out_ref[...] += part


def kernel(keys, vals, rpe, query, W, b):
    kt = jnp.transpose(keys, (0, 2, 1))
    qt = jnp.transpose(query.reshape(B, H * E))
    rpe2 = rpe.reshape(T, B)
    b2 = b.reshape(1, D)

    grid = (B // BB, D // DC)
    return pl.pallas_call(
        _dnd_read_kernel,
        grid=grid,
        in_specs=[
            pl.BlockSpec((T, E, BB), lambda i, s: (0, 0, i)),
            pl.BlockSpec((T, BB, DC), lambda i, s: (0, i, s)),
            pl.BlockSpec((T, BB), lambda i, s: (0, i)),
            pl.BlockSpec((H * E, BB), lambda i, s: (0, i)),
            pl.BlockSpec((D, H * D), lambda i, s: (0, 0)),
            pl.BlockSpec((1, D), lambda i, s: (0, 0)),
        ],
        out_specs=pl.BlockSpec((BB, D), lambda i, s: (i, 0)),
        out_shape=jax.ShapeDtypeStruct((B, D), jnp.float32),
        scratch_shapes=[
            pltpu.VMEM((T, BB), jnp.float32),
            pltpu.VMEM((T, BB), jnp.float32),
        ],
    )(kt, vals, rpe2, qt, W, b2)
